# baseline (speedup 1.0000x reference)
import math

import jax
import jax.numpy as jnp
from jax import lax
from jax.experimental import pallas as pl
from jax.experimental.pallas import tpu as pltpu
from jax.experimental.pallas import tpu_sc as plsc

DIM = 64
SCALE = math.sqrt(DIM)
LANES = 16
CHUNK = 128
TILE_WORDS = 8 * CHUNK


def _make_kernel(hist: int, num_workers: int, steps: int):
    n_btiles = steps * num_workers // hist
    mesh = plsc.VectorSubcoreMesh(core_axis_name="c", subcore_axis_name="s")

    def body(idx_hbm, table_hbm, out_hbm, idx_v, g0, g1, t0, t1,
             sg0, sg1, st0, st1):
        nc = mesh.num_cores
        wid = lax.axis_index("s") * nc + lax.axis_index("c")
        beta0 = wid * steps
        gb = (g0, g1)
        tb = (t0, t1)
        sg = (sg0, sg1)
        st = (st0, st1)

        pltpu.sync_copy(idx_hbm.at[wid], idx_v)

        lane_iota = lax.iota(jnp.int32, LANES)
        scat = [lane_iota * CHUNK + (q * LANES * CHUNK)
                for q in range(DIM // LANES)]

        def start_gather(j, b):
            pltpu.async_copy(table_hbm.at[idx_v.at[j]], gb[b], sg[b])

        def transpose_scale(b):
            src = gb[b]
            dst = tb[b]

            @pl.loop(0, CHUNK)
            def _(t):
                for q in range(DIM // LANES):
                    vals = src[t, pl.ds(q * LANES, LANES)]
                    plsc.store_scatter(dst, [scat[q] + t], vals * SCALE)

        def drain_out(b):
            for dg in range(DIM // 8):
                pltpu.make_async_copy(
                    tb[b].at[pl.ds(dg * TILE_WORDS, TILE_WORDS)],
                    out_hbm.at[0, dg, 0], st[b]).wait()

        def pipe_step(j, b, *, out_wait, prefetch):
            beta = beta0 + j
            h = beta // n_btiles
            bt = beta - h * n_btiles
            pltpu.make_async_copy(table_hbm.at[idx_v.at[j]], gb[b], sg[b]).wait()
            if out_wait:
                drain_out(b)
            transpose_scale(b)
            if prefetch:
                start_gather(j + 2, b)
            for dg in range(DIM // 8):
                pltpu.async_copy(
                    tb[b].at[pl.ds(dg * TILE_WORDS, TILE_WORDS)],
                    out_hbm.at[h, dg, bt], st[b])

        start_gather(0, 0)
        start_gather(1, 1)
        pipe_step(0, 0, out_wait=False, prefetch=True)
        pipe_step(1, 1, out_wait=False, prefetch=True)

        @pl.loop(1, steps // 2 - 1)
        def _(g):
            pipe_step(2 * g, 0, out_wait=True, prefetch=True)
            pipe_step(2 * g + 1, 1, out_wait=True, prefetch=True)

        pipe_step(steps - 2, 0, out_wait=True, prefetch=False)
        pipe_step(steps - 1, 1, out_wait=True, prefetch=False)
        drain_out(0)
        drain_out(1)

    kern = pl.kernel(
        body,
        out_type=jax.ShapeDtypeStruct(
            (hist, DIM // 8, n_btiles, TILE_WORDS), jnp.float32),
        mesh=mesh,
        compiler_params=pltpu.CompilerParams(
            use_tc_tiling_on_sc=False, needs_layout_passes=False),
        scratch_types=[
            pltpu.VMEM((steps, CHUNK), jnp.int32),
            pltpu.VMEM((CHUNK, DIM), jnp.float32),
            pltpu.VMEM((CHUNK, DIM), jnp.float32),
            pltpu.VMEM((DIM * CHUNK,), jnp.float32),
            pltpu.VMEM((DIM * CHUNK,), jnp.float32),
            pltpu.SemaphoreType.DMA,
            pltpu.SemaphoreType.DMA,
            pltpu.SemaphoreType.DMA,
            pltpu.SemaphoreType.DMA,
        ],
    )
    return kern


def kernel(x, w_embed):
    batch, hist = x.shape
    total = batch * hist
    info = plsc.get_sparse_core_info()
    num_workers = info.num_cores * info.num_subcores
    steps = total // (num_workers * CHUNK)
    assert steps * num_workers * CHUNK == total
    assert batch % CHUNK == 0
    idx = x.T.reshape(num_workers, steps, CHUNK).astype(jnp.int32)
    out4 = _make_kernel(hist, num_workers, steps)(idx, w_embed)
    out5 = out4.reshape(hist, DIM // 8, batch // CHUNK, 8, CHUNK)
    out = out5.transpose(2, 4, 0, 1, 3).reshape(batch, hist, DIM)
    return out


# speedup vs baseline: 1.0155x; 1.0155x over previous
import math

import jax
import jax.numpy as jnp
from jax import lax
from jax.experimental import pallas as pl
from jax.experimental.pallas import tpu as pltpu
from jax.experimental.pallas import tpu_sc as plsc

DIM = 64
SCALE = math.sqrt(DIM)
LANES = 16
CHUNK = 128
TILE_WORDS = 8 * CHUNK


def _make_kernel(hist: int, num_workers: int, steps: int):
    n_btiles = steps * num_workers // hist
    mesh = plsc.VectorSubcoreMesh(core_axis_name="c", subcore_axis_name="s")

    def body(idx_hbm, table_hbm, out_hbm, idx_v, g0, g1, t0, t1,
             sg0, sg1, st0, st1):
        nc = mesh.num_cores
        wid = lax.axis_index("s") * nc + lax.axis_index("c")
        beta0 = wid * steps
        gb = (g0, g1)
        tb = (t0, t1)
        sg = (sg0, sg1)
        st = (st0, st1)

        pltpu.sync_copy(idx_hbm.at[wid], idx_v)

        lane_iota = lax.iota(jnp.int32, LANES)
        scat_row = [(lane_iota + q * LANES) // 8 for q in range(DIM // LANES)]
        scat_col = [((lane_iota + q * LANES) % 8) * CHUNK
                    for q in range(DIM // LANES)]

        def start_gather(j, b):
            pltpu.async_copy(table_hbm.at[idx_v.at[j]], gb[b], sg[b])

        def transpose_scale(b):
            src = gb[b]
            dst = tb[b]

            @pl.loop(0, CHUNK, unroll=4)
            def _(t):
                for q in range(DIM // LANES):
                    vals = src[t, pl.ds(q * LANES, LANES)]
                    plsc.store_scatter(
                        dst, [scat_row[q], scat_col[q] + t], vals * SCALE)

        def drain_out(b):
            pltpu.make_async_copy(tb[b], out_hbm.at[0, :, 0], st[b]).wait()

        def pipe_step(j, b, *, out_wait, prefetch):
            beta = beta0 + j
            h = beta // n_btiles
            bt = beta - h * n_btiles
            pltpu.make_async_copy(table_hbm.at[idx_v.at[j]], gb[b], sg[b]).wait()
            if out_wait:
                drain_out(b)
            transpose_scale(b)
            if prefetch:
                start_gather(j + 2, b)
            pltpu.async_copy(tb[b], out_hbm.at[h, :, bt], st[b])

        start_gather(0, 0)
        start_gather(1, 1)
        pipe_step(0, 0, out_wait=False, prefetch=True)
        pipe_step(1, 1, out_wait=False, prefetch=True)

        @pl.loop(1, steps // 2 - 1)
        def _(g):
            pipe_step(2 * g, 0, out_wait=True, prefetch=True)
            pipe_step(2 * g + 1, 1, out_wait=True, prefetch=True)

        pipe_step(steps - 2, 0, out_wait=True, prefetch=False)
        pipe_step(steps - 1, 1, out_wait=True, prefetch=False)
        drain_out(0)
        drain_out(1)

    kern = pl.kernel(
        body,
        out_type=jax.ShapeDtypeStruct(
            (hist, DIM // 8, n_btiles, TILE_WORDS), jnp.float32),
        mesh=mesh,
        compiler_params=pltpu.CompilerParams(
            use_tc_tiling_on_sc=False, needs_layout_passes=False),
        scratch_types=[
            pltpu.VMEM((steps, CHUNK), jnp.int32),
            pltpu.VMEM((CHUNK, DIM), jnp.float32),
            pltpu.VMEM((CHUNK, DIM), jnp.float32),
            pltpu.VMEM((DIM // 8, 8 * CHUNK), jnp.float32),
            pltpu.VMEM((DIM // 8, 8 * CHUNK), jnp.float32),
            pltpu.SemaphoreType.DMA,
            pltpu.SemaphoreType.DMA,
            pltpu.SemaphoreType.DMA,
            pltpu.SemaphoreType.DMA,
        ],
    )
    return kern


def kernel(x, w_embed):
    batch, hist = x.shape
    total = batch * hist
    info = plsc.get_sparse_core_info()
    num_workers = info.num_cores * info.num_subcores
    steps = total // (num_workers * CHUNK)
    assert steps * num_workers * CHUNK == total
    assert batch % CHUNK == 0
    idx = x.T.reshape(num_workers, steps, CHUNK).astype(jnp.int32)
    out4 = _make_kernel(hist, num_workers, steps)(idx, w_embed)
    out5 = out4.reshape(hist, DIM // 8, batch // CHUNK, 8, CHUNK)
    out = out5.transpose(2, 4, 0, 1, 3).reshape(batch, hist, DIM)
    return out


# speedup vs baseline: 1.5761x; 1.5520x over previous
import math

import jax
import jax.numpy as jnp
from jax import lax
from jax.experimental import pallas as pl
from jax.experimental.pallas import tpu as pltpu
from jax.experimental.pallas import tpu_sc as plsc

DIM = 64
SCALE = math.sqrt(DIM)
LANES = 16
CHUNK = 128
PADC = CHUNK + 5


def _make_kernel(hist: int, num_workers: int, steps: int):
    n_btiles = steps * num_workers // hist
    mesh = plsc.VectorSubcoreMesh(core_axis_name="c", subcore_axis_name="s")

    def body(idx_hbm, table_hbm, out_hbm, idx_v, g0, g1, t0, t1,
             sg0, sg1, st0, st1):
        nc = mesh.num_cores
        wid = lax.axis_index("s") * nc + lax.axis_index("c")
        beta0 = wid * steps
        gb = (g0, g1)
        tb = (t0, t1)
        sg = (sg0, sg1)
        st = (st0, st1)

        pltpu.sync_copy(idx_hbm.at[wid], idx_v)

        lane_iota = lax.iota(jnp.int32, LANES)
        scat_dg = [(lane_iota + q * LANES) // 8 for q in range(DIM // LANES)]
        scat_di = [(lane_iota + q * LANES) % 8 for q in range(DIM // LANES)]

        def start_gather(j, b):
            pltpu.async_copy(table_hbm.at[idx_v.at[j]], gb[b], sg[b])

        def transpose_scale(b):
            src = gb[b]
            dst = tb[b]

            @pl.loop(0, CHUNK, unroll=4)
            def _(t):
                t_vec = lane_iota * 0 + t
                for q in range(DIM // LANES):
                    vals = src[t, pl.ds(q * LANES, LANES)]
                    plsc.store_scatter(
                        dst, [scat_dg[q], scat_di[q], t_vec], vals * SCALE)

        def drain_out(b):
            pltpu.make_async_copy(
                tb[b].at[:, :, pl.ds(0, CHUNK)], out_hbm.at[0, :, 0],
                st[b]).wait()

        def pipe_step(j, b, *, out_wait, prefetch):
            beta = beta0 + j
            h = beta // n_btiles
            bt = beta - h * n_btiles
            pltpu.make_async_copy(table_hbm.at[idx_v.at[j]], gb[b], sg[b]).wait()
            if out_wait:
                drain_out(b)
            transpose_scale(b)
            if prefetch:
                start_gather(j + 2, b)
            pltpu.async_copy(
                tb[b].at[:, :, pl.ds(0, CHUNK)], out_hbm.at[h, :, bt], st[b])

        start_gather(0, 0)
        start_gather(1, 1)
        pipe_step(0, 0, out_wait=False, prefetch=True)
        pipe_step(1, 1, out_wait=False, prefetch=True)

        @pl.loop(1, steps // 2 - 1)
        def _(g):
            pipe_step(2 * g, 0, out_wait=True, prefetch=True)
            pipe_step(2 * g + 1, 1, out_wait=True, prefetch=True)

        pipe_step(steps - 2, 0, out_wait=True, prefetch=False)
        pipe_step(steps - 1, 1, out_wait=True, prefetch=False)
        drain_out(0)
        drain_out(1)

    kern = pl.kernel(
        body,
        out_type=jax.ShapeDtypeStruct(
            (hist, DIM // 8, n_btiles, 8, CHUNK), jnp.float32),
        mesh=mesh,
        compiler_params=pltpu.CompilerParams(
            use_tc_tiling_on_sc=False, needs_layout_passes=False),
        scratch_types=[
            pltpu.VMEM((steps, CHUNK), jnp.int32),
            pltpu.VMEM((CHUNK, DIM), jnp.float32),
            pltpu.VMEM((CHUNK, DIM), jnp.float32),
            pltpu.VMEM((DIM // 8, 8, PADC), jnp.float32),
            pltpu.VMEM((DIM // 8, 8, PADC), jnp.float32),
            pltpu.SemaphoreType.DMA,
            pltpu.SemaphoreType.DMA,
            pltpu.SemaphoreType.DMA,
            pltpu.SemaphoreType.DMA,
        ],
    )
    return kern


def kernel(x, w_embed):
    batch, hist = x.shape
    total = batch * hist
    info = plsc.get_sparse_core_info()
    num_workers = info.num_cores * info.num_subcores
    steps = total // (num_workers * CHUNK)
    assert steps * num_workers * CHUNK == total
    assert batch % CHUNK == 0
    idx = x.T.reshape(num_workers, steps, CHUNK).astype(jnp.int32)
    out5 = _make_kernel(hist, num_workers, steps)(idx, w_embed)
    out = out5.transpose(2, 4, 0, 1, 3).reshape(batch, hist, DIM)
    return out
